# baseline (speedup 1.0000x reference)
import functools

import jax
import jax.numpy as jnp
from jax import lax
from jax.experimental import pallas as pl
from jax.experimental.pallas import tpu as pltpu
from jax.experimental.pallas import tpu_sc as plsc

_CHUNK = 128


def _make_sc_neighbor_sum(n_nodes, half_d, chunks_per_tile,
                          acc_rows, num_cores, num_subcores):
    zr = acc_rows // num_subcores
    w_tiles = 10
    wr = n_nodes // w_tiles
    cpt = chunks_per_tile

    mesh = plsc.VectorSubcoreMesh(core_axis_name="c", subcore_axis_name="s")
    nb = 16

    @functools.partial(
        pl.kernel,
        out_type=jax.ShapeDtypeStruct((2 * n_nodes, half_d), jnp.float32),
        mesh=mesh,
        scratch_types=[
            pltpu.VMEM((nb, _CHUNK), jnp.int32),
            pltpu.VMEM((nb, _CHUNK), jnp.int32),
            pltpu.VMEM((_CHUNK, half_d), jnp.float32),
            pltpu.VMEM_SHARED((acc_rows, half_d), jnp.float32),
            pltpu.SemaphoreType.DMA,
        ],
    )
    def sc_kernel(h_hbm, col_hbm, row_hbm, zero_hbm, out_hbm,
                  colv, rowv, gbuf, acc, sem):
        c = lax.axis_index("c")
        s = lax.axis_index("s")

        off = c * n_nodes

        pltpu.sync_copy(zero_hbm.at[pl.ds(s * zr, zr)], acc.at[pl.ds(s * zr, zr)])
        plsc.subcore_barrier()

        def block(b, carry):
            base = s * cpt + b * nb
            pltpu.sync_copy(col_hbm.at[pl.ds(base, nb)], colv)
            pltpu.sync_copy(row_hbm.at[pl.ds(base, nb)], rowv)

            def adj(r, cy):
                for k in range(_CHUNK // 16):
                    sl = pl.ds(k * 16, 16)
                    colv[r, sl] = colv[r, sl] + off
                return cy

            lax.fori_loop(0, nb, adj, 0)

            def step(j, cy):
                pltpu.async_copy(h_hbm.at[colv.at[j]], gbuf, sem).wait()
                pltpu.sync_copy(gbuf, acc.at[rowv.at[j]], add=True)
                return cy

            lax.fori_loop(0, nb, step, 0)
            return carry

        lax.fori_loop(0, cpt // nb, block, 0)
        plsc.subcore_barrier()

        @pl.when(s < w_tiles)
        def _():
            pltpu.sync_copy(acc.at[pl.ds(s * wr, wr)],
                            out_hbm.at[pl.ds(c * n_nodes + s * wr, wr)])

    return sc_kernel


_MAXT = 256


def _build_edge_layout(edge_index, n, n_edges, num_subcores, cpt):
    row = edge_index[0]
    col = edge_index[1]
    perm = jnp.argsort(row)
    srow = row[perm]
    scol = col[perm]
    deg = jnp.zeros((n,), jnp.int32).at[row].add(1)
    start = jnp.cumsum(deg) - deg
    raw = (jnp.arange(num_subcores) * n_edges) // num_subcores
    cnode = srow[raw].at[0].set(0)
    node_tile = (jnp.searchsorted(cnode, jnp.arange(n), side="right") - 1
                 ).astype(jnp.int32)
    k = jnp.arange(n_edges, dtype=jnp.int32)
    t_k = jnp.minimum(k - start[srow], _MAXT - 1)
    tile_k = node_tile[srow]
    key = tile_k * _MAXT + t_k
    counts = jnp.zeros((num_subcores * _MAXT,), jnp.int32).at[key].add(1)
    rc = -(-counts.reshape(num_subcores, _MAXT) // _CHUNK)
    roff = jnp.cumsum(rc, axis=1) - rc
    order = jnp.argsort(key)
    skey = key[order]
    rank = jnp.arange(n_edges, dtype=jnp.int32) - jnp.searchsorted(
        skey, skey, side="left").astype(jnp.int32)
    chunk = roff.reshape(-1)[skey] + rank // _CHUNK
    slot = (skey // _MAXT) * (cpt * _CHUNK) + chunk * _CHUNK + rank % _CHUNK
    total = num_subcores * cpt * _CHUNK
    slot = jnp.where(chunk < cpt, slot, total)
    rows = jnp.full((total,), n, jnp.int32).at[slot].set(srow[order])
    cols = jnp.zeros((total,), jnp.int32).at[slot].set(scol[order])
    return rows.reshape(-1, _CHUNK), cols.reshape(-1, _CHUNK)


def _make_tc_layer(n_nodes, d_in, hid, fp_dim, ng, row_block):
    grid = (n_nodes // row_block,)
    half = d_in // 2
    dn = (((1,), (1,)), ((), ()))
    hp = lax.Precision.HIGHEST

    def tc_body(h_ref, n_ref, ws_ref, wn_ref, b_ref, wfp_ref, batch_ref,
                fpin_ref, hout_ref, fpout_ref):
        i = pl.program_id(0)
        h_full = jnp.concatenate([h_ref[0], h_ref[1]], axis=1)
        n_full = jnp.concatenate([n_ref[0], n_ref[1]], axis=1)
        z = (lax.dot_general(h_full, ws_ref[...], dn,
                             preferred_element_type=jnp.float32)
             + lax.dot_general(n_full, wn_ref[...], dn,
                               preferred_element_type=jnp.float32)
             + b_ref[...])
        hn = jnp.tanh(z)
        hout_ref[0] = hn[:, :half]
        hout_ref[1] = hn[:, half:]
        s = lax.dot_general(hn, wfp_ref[...], dn,
                            preferred_element_type=jnp.float32)
        m = jnp.max(s, axis=1, keepdims=True)
        e = jnp.exp(s - m)
        contrib = e / jnp.sum(e, axis=1, keepdims=True)
        bids = batch_ref[0, 0, :]
        onehot = (bids[None, :] ==
                  lax.broadcasted_iota(jnp.int32, (ng, row_block), 0)
                  ).astype(jnp.float32)
        part = lax.dot_general(onehot, contrib, (((1,), (0,)), ((), ())),
                               precision=hp,
                               preferred_element_type=jnp.float32)

        @pl.when(i == 0)
        def _():
            fpout_ref[...] = fpin_ref[...]

        fpout_ref[...] += part

    return pl.pallas_call(
        tc_body,
        grid=grid,
        in_specs=[
            pl.BlockSpec((2, row_block, half), lambda i: (0, i, 0)),
            pl.BlockSpec((2, row_block, half), lambda i: (0, i, 0)),
            pl.BlockSpec((hid, d_in), lambda i: (0, 0)),
            pl.BlockSpec((hid, d_in), lambda i: (0, 0)),
            pl.BlockSpec((1, hid), lambda i: (0, 0)),
            pl.BlockSpec((fp_dim, hid), lambda i: (0, 0)),
            pl.BlockSpec((1, 1, row_block), lambda i: (i, 0, 0)),
            pl.BlockSpec((ng, fp_dim), lambda i: (0, 0)),
        ],
        out_specs=[
            pl.BlockSpec((2, row_block, half), lambda i: (0, i, 0)),
            pl.BlockSpec((ng, fp_dim), lambda i: (0, 0)),
        ],
        out_shape=[
            jax.ShapeDtypeStruct((2, n_nodes, half), jnp.float32),
            jax.ShapeDtypeStruct((ng, fp_dim), jnp.float32),
        ],
        compiler_params=pltpu.CompilerParams(
            dimension_semantics=("arbitrary",)),
    )


def kernel(x, edge_index, batch, W_self, b_self, W_neigh, b_neigh, W_fp):
    n, d_in = x.shape
    n_edges = edge_index.shape[1]
    layers, hid, _ = W_self.shape
    fp_dim = W_fp.shape[1]
    ng = 64
    half = d_in // 2

    info = plsc.get_sparse_core_info()
    num_cores, num_subcores = info.num_cores, info.num_subcores

    cpt = 8 * (-(-((-(-n_edges // (num_subcores * _CHUNK))) + 64) // 8))
    acc_rows = 8 * num_subcores * (-(-(n + 1) // (8 * num_subcores)))
    row2d, col2d = _build_edge_layout(edge_index, n, n_edges, num_subcores, cpt)
    zeros_init = jnp.zeros((acc_rows, half), jnp.float32)

    sc_neighbor_sum = _make_sc_neighbor_sum(
        n, half, cpt, acc_rows, num_cores, num_subcores)
    tc_layer = _make_tc_layer(n, d_in, hid, fp_dim, ng, row_block=2000)

    h2 = jnp.concatenate([x[:, :half], x[:, half:]], axis=0)
    batch3 = batch.reshape(n // 2000, 1, 2000)
    fp = jnp.zeros((ng, fp_dim), jnp.float32)

    for l in range(layers):
        neigh = sc_neighbor_sum(h2, col2d, row2d, zeros_init)
        bias = (b_self[l] + b_neigh[l]).reshape(1, hid)
        h3, fp = tc_layer(h2.reshape(2, n, half), neigh.reshape(2, n, half),
                          W_self[l], W_neigh[l], bias, W_fp[l], batch3, fp)
        h2 = h3.reshape(2 * n, half)

    return fp


# speedup vs baseline: 1.0007x; 1.0007x over previous
import functools

import jax
import jax.numpy as jnp
from jax import lax
from jax.experimental import pallas as pl
from jax.experimental.pallas import tpu as pltpu
from jax.experimental.pallas import tpu_sc as plsc

_CHUNK = 128


def _make_sc_neighbor_sum(n_nodes, half_d, chunks_per_tile,
                          acc_rows, num_cores, num_subcores):
    zr = acc_rows // num_subcores
    w_tiles = 10
    wr = n_nodes // w_tiles
    cpt = chunks_per_tile

    mesh = plsc.VectorSubcoreMesh(core_axis_name="c", subcore_axis_name="s")
    nb = 8

    @functools.partial(
        pl.kernel,
        out_type=jax.ShapeDtypeStruct((2 * n_nodes, half_d), jnp.float32),
        mesh=mesh,
        scratch_types=[
            pltpu.VMEM((nb, _CHUNK), jnp.int32),
            pltpu.VMEM((nb, _CHUNK), jnp.int32),
            pltpu.VMEM((2, _CHUNK, half_d), jnp.float32),
            pltpu.VMEM_SHARED((acc_rows, half_d), jnp.float32),
            pltpu.SemaphoreType.DMA,
        ],
    )
    def sc_kernel(h_hbm, col_hbm, row_hbm, zero_hbm, out_hbm,
                  colv, rowv, gbuf, acc, sem):
        c = lax.axis_index("c")
        s = lax.axis_index("s")

        off = c * n_nodes

        pltpu.sync_copy(zero_hbm.at[pl.ds(s * zr, zr)], acc.at[pl.ds(s * zr, zr)])
        plsc.subcore_barrier()

        def block(b, carry):
            base = s * cpt + b * nb
            pltpu.sync_copy(col_hbm.at[pl.ds(base, nb)], colv)
            pltpu.sync_copy(row_hbm.at[pl.ds(base, nb)], rowv)

            def adj(r, cy):
                for k in range(_CHUNK // 16):
                    sl = pl.ds(k * 16, 16)
                    colv[r, sl] = colv[r, sl] + off
                return cy

            lax.fori_loop(0, nb, adj, 0)

            pltpu.async_copy(h_hbm.at[colv.at[0]], gbuf.at[0], sem)
            for j in range(nb):
                pltpu.make_async_copy(h_hbm.at[colv.at[j]],
                                      gbuf.at[j % 2], sem).wait()
                if j + 1 < nb:
                    pltpu.async_copy(h_hbm.at[colv.at[j + 1]],
                                     gbuf.at[(j + 1) % 2], sem)
                pltpu.sync_copy(gbuf.at[j % 2], acc.at[rowv.at[j]], add=True)
            return carry

        lax.fori_loop(0, cpt // nb, block, 0)
        plsc.subcore_barrier()

        @pl.when(s < w_tiles)
        def _():
            pltpu.sync_copy(acc.at[pl.ds(s * wr, wr)],
                            out_hbm.at[pl.ds(c * n_nodes + s * wr, wr)])

    return sc_kernel


_MAXT = 256


def _build_edge_layout(edge_index, n, n_edges, num_subcores, cpt):
    row = edge_index[0]
    col = edge_index[1]
    perm = jnp.argsort(row)
    srow = row[perm]
    scol = col[perm]
    deg = jnp.zeros((n,), jnp.int32).at[row].add(1)
    start = jnp.cumsum(deg) - deg
    raw = (jnp.arange(num_subcores) * n_edges) // num_subcores
    cnode = srow[raw].at[0].set(0)
    node_tile = (jnp.searchsorted(cnode, jnp.arange(n), side="right") - 1
                 ).astype(jnp.int32)
    k = jnp.arange(n_edges, dtype=jnp.int32)
    t_k = jnp.minimum(k - start[srow], _MAXT - 1)
    tile_k = node_tile[srow]
    key = tile_k * _MAXT + t_k
    counts = jnp.zeros((num_subcores * _MAXT,), jnp.int32).at[key].add(1)
    rc = -(-counts.reshape(num_subcores, _MAXT) // _CHUNK)
    roff = jnp.cumsum(rc, axis=1) - rc
    order = jnp.argsort(key)
    skey = key[order]
    rank = jnp.arange(n_edges, dtype=jnp.int32) - jnp.searchsorted(
        skey, skey, side="left").astype(jnp.int32)
    chunk = roff.reshape(-1)[skey] + rank // _CHUNK
    slot = (skey // _MAXT) * (cpt * _CHUNK) + chunk * _CHUNK + rank % _CHUNK
    total = num_subcores * cpt * _CHUNK
    slot = jnp.where(chunk < cpt, slot, total)
    rows = jnp.full((total,), n, jnp.int32).at[slot].set(srow[order])
    cols = jnp.zeros((total,), jnp.int32).at[slot].set(scol[order])
    return rows.reshape(-1, _CHUNK), cols.reshape(-1, _CHUNK)


def _make_tc_layer(n_nodes, d_in, hid, fp_dim, ng, row_block):
    grid = (n_nodes // row_block,)
    half = d_in // 2
    dn = (((1,), (1,)), ((), ()))
    hp = lax.Precision.HIGHEST

    def tc_body(h_ref, n_ref, ws_ref, wn_ref, b_ref, wfp_ref, batch_ref,
                fpin_ref, hout_ref, fpout_ref):
        i = pl.program_id(0)
        h_full = jnp.concatenate([h_ref[0], h_ref[1]], axis=1)
        n_full = jnp.concatenate([n_ref[0], n_ref[1]], axis=1)
        z = (lax.dot_general(h_full, ws_ref[...], dn,
                             preferred_element_type=jnp.float32)
             + lax.dot_general(n_full, wn_ref[...], dn,
                               preferred_element_type=jnp.float32)
             + b_ref[...])
        hn = jnp.tanh(z)
        hout_ref[0] = hn[:, :half]
        hout_ref[1] = hn[:, half:]
        s = lax.dot_general(hn, wfp_ref[...], dn,
                            preferred_element_type=jnp.float32)
        m = jnp.max(s, axis=1, keepdims=True)
        e = jnp.exp(s - m)
        contrib = e / jnp.sum(e, axis=1, keepdims=True)
        bids = batch_ref[0, 0, :]
        onehot = (bids[None, :] ==
                  lax.broadcasted_iota(jnp.int32, (ng, row_block), 0)
                  ).astype(jnp.float32)
        part = lax.dot_general(onehot, contrib, (((1,), (0,)), ((), ())),
                               precision=hp,
                               preferred_element_type=jnp.float32)

        @pl.when(i == 0)
        def _():
            fpout_ref[...] = fpin_ref[...]

        fpout_ref[...] += part

    return pl.pallas_call(
        tc_body,
        grid=grid,
        in_specs=[
            pl.BlockSpec((2, row_block, half), lambda i: (0, i, 0)),
            pl.BlockSpec((2, row_block, half), lambda i: (0, i, 0)),
            pl.BlockSpec((hid, d_in), lambda i: (0, 0)),
            pl.BlockSpec((hid, d_in), lambda i: (0, 0)),
            pl.BlockSpec((1, hid), lambda i: (0, 0)),
            pl.BlockSpec((fp_dim, hid), lambda i: (0, 0)),
            pl.BlockSpec((1, 1, row_block), lambda i: (i, 0, 0)),
            pl.BlockSpec((ng, fp_dim), lambda i: (0, 0)),
        ],
        out_specs=[
            pl.BlockSpec((2, row_block, half), lambda i: (0, i, 0)),
            pl.BlockSpec((ng, fp_dim), lambda i: (0, 0)),
        ],
        out_shape=[
            jax.ShapeDtypeStruct((2, n_nodes, half), jnp.float32),
            jax.ShapeDtypeStruct((ng, fp_dim), jnp.float32),
        ],
        compiler_params=pltpu.CompilerParams(
            dimension_semantics=("arbitrary",)),
    )


def kernel(x, edge_index, batch, W_self, b_self, W_neigh, b_neigh, W_fp):
    n, d_in = x.shape
    n_edges = edge_index.shape[1]
    layers, hid, _ = W_self.shape
    fp_dim = W_fp.shape[1]
    ng = 64
    half = d_in // 2

    info = plsc.get_sparse_core_info()
    num_cores, num_subcores = info.num_cores, info.num_subcores

    cpt = 8 * (-(-((-(-n_edges // (num_subcores * _CHUNK))) + 64) // 8))
    acc_rows = 8 * num_subcores * (-(-(n + 1) // (8 * num_subcores)))
    row2d, col2d = _build_edge_layout(edge_index, n, n_edges, num_subcores, cpt)
    zeros_init = jnp.zeros((acc_rows, half), jnp.float32)

    sc_neighbor_sum = _make_sc_neighbor_sum(
        n, half, cpt, acc_rows, num_cores, num_subcores)
    tc_layer = _make_tc_layer(n, d_in, hid, fp_dim, ng, row_block=2000)

    h2 = jnp.concatenate([x[:, :half], x[:, half:]], axis=0)
    batch3 = batch.reshape(n // 2000, 1, 2000)
    fp = jnp.zeros((ng, fp_dim), jnp.float32)

    for l in range(layers):
        neigh = sc_neighbor_sum(h2, col2d, row2d, zeros_init)
        bias = (b_self[l] + b_neigh[l]).reshape(1, hid)
        h3, fp = tc_layer(h2.reshape(2, n, half), neigh.reshape(2, n, half),
                          W_self[l], W_neigh[l], bias, W_fp[l], batch3, fp)
        h2 = h3.reshape(2 * n, half)

    return fp
